# Initial kernel scaffold; baseline (speedup 1.0000x reference)
import functools

import jax
import jax.numpy as jnp
from jax import lax
from jax.experimental import pallas as pl
from jax.experimental.pallas import tpu as pltpu
from jax.experimental.pallas import tpu_sc as plsc

_NUM_CORES = 2
_NUM_SUBCORES = 16
_NUM_WORKERS = _NUM_CORES * _NUM_SUBCORES

_GATHER_ROWS = 128


def _emb_body(ng, idx_hbm, table_hbm, out_hbm, idx_v, rows_v, sem):
  wid = lax.axis_index("s") * _NUM_CORES + lax.axis_index("c")
  pltpu.sync_copy(idx_hbm.at[wid], idx_v)

  def chunk(j, _):
    pltpu.async_copy(table_hbm.at[idx_v.at[j]], rows_v, sem).wait()
    pltpu.sync_copy(rows_v, out_hbm.at[wid, j])
    return 0

  lax.fori_loop(0, ng, chunk, 0)


def kernel(word_ids, table):
  n_tok, seq = word_ids.shape
  vocab, d = table.shape
  b = n_tok * seq
  assert b % (_NUM_WORKERS * _GATHER_ROWS) == 0
  ng = b // (_NUM_WORKERS * _GATHER_ROWS)

  idx = word_ids.reshape(_NUM_WORKERS, ng, _GATHER_ROWS).astype(jnp.int32)

  mesh = plsc.VectorSubcoreMesh(
      core_axis_name="c", subcore_axis_name="s",
      num_cores=_NUM_CORES, num_subcores=_NUM_SUBCORES)

  run = pl.kernel(
      functools.partial(_emb_body, ng),
      out_type=jax.ShapeDtypeStruct(
          (_NUM_WORKERS, ng, _GATHER_ROWS, d), jnp.float32),
      mesh=mesh,
      scratch_types=[
          pltpu.VMEM((ng, _GATHER_ROWS), jnp.int32),
          pltpu.VMEM((_GATHER_ROWS, d), jnp.float32),
          pltpu.SemaphoreType.DMA,
      ],
  )
  out = run(idx, table)
  return out.reshape(n_tok, seq, d)


# speedup vs baseline: 1.6831x; 1.6831x over previous
import functools

import jax
import jax.numpy as jnp
from jax import lax
from jax.experimental import pallas as pl
from jax.experimental.pallas import tpu as pltpu
from jax.experimental.pallas import tpu_sc as plsc

_NUM_CORES = 2
_NUM_SUBCORES = 16
_NUM_WORKERS = _NUM_CORES * _NUM_SUBCORES

_GATHER_ROWS = 128


def _emb_body(ng, idx_hbm, table_hbm, out_hbm, idx_v, rows_v, sem):
  wid = lax.axis_index("s") * _NUM_CORES + lax.axis_index("c")
  pltpu.sync_copy(idx_hbm.at[wid], idx_v)

  def chunk(j, _):
    pltpu.async_copy(table_hbm.at[idx_v.at[j]], rows_v, sem).wait()
    pltpu.sync_copy(rows_v, out_hbm.at[wid, j])
    return 0

  lax.fori_loop(0, ng, chunk, 0)


def kernel(word_ids, table):
  n_tok, seq = word_ids.shape
  vocab, d = table.shape
  b = n_tok * seq
  assert b % (_NUM_WORKERS * _GATHER_ROWS) == 0
  ng = b // (_NUM_WORKERS * _GATHER_ROWS)

  idx = word_ids.reshape(_NUM_WORKERS, ng, _GATHER_ROWS).astype(jnp.int32)

  mesh = plsc.VectorSubcoreMesh(
      core_axis_name="c", subcore_axis_name="s",
      num_cores=_NUM_CORES, num_subcores=_NUM_SUBCORES)

  run = pl.kernel(
      functools.partial(_emb_body, ng),
      out_type=jax.ShapeDtypeStruct(
          (_NUM_WORKERS, ng, _GATHER_ROWS, d), jnp.float32),
      mesh=mesh,
      scratch_types=[
          pltpu.VMEM((ng, _GATHER_ROWS), jnp.int32),
          pltpu.VMEM((_GATHER_ROWS, d), jnp.float32),
          pltpu.SemaphoreType.DMA,
      ],
      compiler_params=pltpu.CompilerParams(use_tc_tiling_on_sc=False),
  )
  out = run(idx, table)
  return out.reshape(n_tok, seq, d)


# speedup vs baseline: 1.8762x; 1.1147x over previous
import functools

import jax
import jax.numpy as jnp
from jax import lax
from jax.experimental import pallas as pl
from jax.experimental.pallas import tpu as pltpu
from jax.experimental.pallas import tpu_sc as plsc

_NUM_CORES = 2
_NUM_SUBCORES = 16
_NUM_WORKERS = _NUM_CORES * _NUM_SUBCORES

_GATHER_ROWS = 128
_NBUF = 4


def _emb_body(ng, idx_hbm, table_hbm, out_hbm, idx_v, rows_v, *sems):
  n = _NBUF
  h = n // 2
  gsem = sems[:n]
  ssem = sems[n:]
  wid = lax.axis_index("s") * _NUM_CORES + lax.axis_index("c")
  pltpu.sync_copy(idx_hbm.at[wid], idx_v)

  def g_start(c, b):
    pltpu.async_copy(table_hbm.at[idx_v.at[c]], rows_v.at[b], gsem[b])

  def g_wait(c, b):
    pltpu.make_async_copy(
        table_hbm.at[idx_v.at[c]], rows_v.at[b], gsem[b]).wait()

  def s_start(c, b):
    pltpu.async_copy(rows_v.at[b], out_hbm.at[wid, c], ssem[b])

  def s_wait(c, b):
    pltpu.make_async_copy(
        rows_v.at[b], out_hbm.at[wid, c], ssem[b]).wait()

  for b in range(h):
    g_start(b, b)

  for b in range(n):
    g_wait(b, b)
    s_start(b, b)
    b2 = (b + h) % n
    if b2 > b:
      g_start(b2, b2)
    else:
      s_wait(b2, b2)
      g_start(n + b2, b2)

  def sweep(jj, _):
    for b in range(n):
      c = jj * n + b
      g_wait(c, b)
      s_start(c, b)
      b2 = (b + h) % n
      if b2 > b:
        c2 = jj * n + b2
      else:
        c2 = (jj + 1) * n + b2
      s_wait(c2 - n, b2)
      g_start(c2, b2)
    return 0

  lax.fori_loop(1, ng // n - 1, sweep, 0)

  last = ng - n
  for b in range(h):
    g_wait(last + b, b)
    s_start(last + b, b)
    b2 = b + h
    s_wait(last - n + b2, b2)
    g_start(last + b2, b2)
  for b in range(h, n):
    g_wait(last + b, b)
    s_start(last + b, b)
  for b in range(n):
    s_wait(last + b, b)


def kernel(word_ids, table):
  n_tok, seq = word_ids.shape
  vocab, d = table.shape
  b = n_tok * seq
  assert b % (_NUM_WORKERS * _GATHER_ROWS) == 0
  ng = b // (_NUM_WORKERS * _GATHER_ROWS)
  assert ng % _NBUF == 0 and ng // _NBUF >= 2

  idx = word_ids.reshape(_NUM_WORKERS, ng, _GATHER_ROWS).astype(jnp.int32)

  mesh = plsc.VectorSubcoreMesh(
      core_axis_name="c", subcore_axis_name="s",
      num_cores=_NUM_CORES, num_subcores=_NUM_SUBCORES)

  run = pl.kernel(
      functools.partial(_emb_body, ng),
      out_type=jax.ShapeDtypeStruct(
          (_NUM_WORKERS, ng, _GATHER_ROWS, d), jnp.float32),
      mesh=mesh,
      scratch_types=[
          pltpu.VMEM((ng, _GATHER_ROWS), jnp.int32),
          pltpu.VMEM((_NBUF, _GATHER_ROWS, d), jnp.float32),
      ] + [pltpu.SemaphoreType.DMA] * (2 * _NBUF),
      compiler_params=pltpu.CompilerParams(use_tc_tiling_on_sc=False),
  )
  out = run(idx, table)
  return out.reshape(n_tok, seq, d)
